# baseline (speedup 1.0000x reference)
import jax
import jax.numpy as jnp
from jax.experimental import pallas as pl
from jax.experimental.pallas import tpu as pltpu

_SD_PAD = 8


def _mlp_body(m1t_ref, aux_ref, x_ref, o_ref, *, hidden_e, group):
    ht = jax.lax.dot_general(
        m1t_ref[...], x_ref[...],
        (((1,), (1,)), ((), ())),
        preferred_element_type=jnp.float32,
    )
    g = jnp.maximum(ht + aux_ref[:, 0:1], 0.0) * aux_ref[:, 1:2]
    s = g.reshape(hidden_e, group, x_ref.shape[0]).sum(axis=0)
    o_ref[...] = s


def kernel(x, params):
    B, sd = x.shape
    p_rows, hidden = params.shape
    assert p_rows == hidden + 1

    group = 128 // sd
    nrows = B // group
    assert B % group == 0

    w1 = params[:hidden, :sd]
    b1 = params[:hidden, sd]
    b2 = params[0, _SD_PAD]
    w2 = params[hidden, :hidden]

    hidden_e = hidden + 1
    w1e = jnp.concatenate([w1, jnp.zeros((1, sd), jnp.float32)], axis=0)
    b1e = jnp.concatenate([b1, jnp.ones((1,), jnp.float32)])
    w2e = jnp.concatenate([w2, b2[None]])

    eye = jnp.eye(group, dtype=jnp.float32)
    m1t = (w1e[:, None, None, :] * eye[None, :, :, None]).reshape(
        hidden_e * group, group * sd)
    aux = jnp.stack(
        [jnp.broadcast_to(b1e[:, None], (hidden_e, group)).reshape(-1),
         jnp.broadcast_to(w2e[:, None], (hidden_e, group)).reshape(-1)],
        axis=1)

    one = 1.0 + 0.0 * params[0, 0]
    xr = (x * one).reshape(nrows, group * sd)

    r_tile = 2048
    while nrows % r_tile:
        r_tile //= 2
    grid = (nrows // r_tile,)

    out = pl.pallas_call(
        lambda m, a, xx, o: _mlp_body(m, a, xx, o, hidden_e=hidden_e, group=group),
        grid=grid,
        in_specs=[
            pl.BlockSpec((hidden_e * group, group * sd), lambda i: (0, 0)),
            pl.BlockSpec((hidden_e * group, 2), lambda i: (0, 0)),
            pl.BlockSpec((r_tile, group * sd), lambda i: (i, 0)),
        ],
        out_specs=pl.BlockSpec((group, r_tile), lambda i: (0, i)),
        out_shape=jax.ShapeDtypeStruct((group, nrows), jnp.float32),
        compiler_params=pltpu.CompilerParams(
            dimension_semantics=("parallel",),
        ),
    )(m1t, aux, xr)

    return out.T.reshape(B, 1)


# speedup vs baseline: 9.9516x; 9.9516x over previous
import jax
import jax.numpy as jnp
from jax.experimental import pallas as pl
from jax.experimental.pallas import tpu as pltpu

_SD_PAD = 8


def _mlp_body(m1q_ref, aux_ref, b2_ref, x_ref, o_ref, *, hidden, n_dots, s_rows):
    chunks = s_rows // 4
    b2 = b2_ref[0, 0]
    for k in range(n_dots):
        xs = x_ref[pl.ds(k * s_rows, s_rows), :]
        h = jnp.dot(m1q_ref[...], xs, preferred_element_type=jnp.float32)
        g = jnp.maximum(h + aux_ref[:, 0:1], 0.0) * aux_ref[:, 1:2]
        s = g.reshape(hidden, chunks, 128).sum(axis=0)
        o_ref[pl.ds(k * chunks, chunks), :] = s + b2


def kernel(x, params):
    B, sd = x.shape
    p_rows, hidden = params.shape
    assert p_rows == hidden + 1
    lanes = 128
    assert B % (lanes * 64) == 0

    w1 = params[:hidden, :sd]
    b1 = params[:hidden, sd]
    b2 = params[0, _SD_PAD]
    w2 = params[hidden, :hidden]

    nrows = (B // lanes) * sd
    xq = x.reshape(B // lanes, lanes, sd).transpose(0, 2, 1).reshape(nrows, lanes)

    s_rows = 256
    dot_chunks = s_rows // sd
    n_dots = 16
    while (B // lanes) % (n_dots * dot_chunks):
        n_dots //= 2
    step_rows = n_dots * s_rows
    grid = (nrows // step_rows,)

    eye = jnp.eye(dot_chunks, dtype=jnp.float32)
    m1q = (w1[:, None, None, :] * eye[None, :, :, None]).reshape(
        hidden * dot_chunks, dot_chunks * sd)
    aux = jnp.stack(
        [jnp.broadcast_to(b1[:, None], (hidden, dot_chunks)).reshape(-1),
         jnp.broadcast_to(w2[:, None], (hidden, dot_chunks)).reshape(-1)],
        axis=1)
    b2a = jnp.reshape(b2, (1, 1))

    out = pl.pallas_call(
        lambda m, a, b, xx, o: _mlp_body(
            m, a, b, xx, o, hidden=hidden, n_dots=n_dots, s_rows=s_rows),
        grid=grid,
        in_specs=[
            pl.BlockSpec((hidden * dot_chunks, s_rows), lambda i: (0, 0)),
            pl.BlockSpec((hidden * dot_chunks, 2), lambda i: (0, 0)),
            pl.BlockSpec(memory_space=pltpu.MemorySpace.SMEM),
            pl.BlockSpec((step_rows, lanes), lambda i: (i, 0)),
        ],
        out_specs=pl.BlockSpec((step_rows // sd, lanes), lambda i: (i, 0)),
        out_shape=jax.ShapeDtypeStruct((B // lanes, lanes), jnp.float32),
        compiler_params=pltpu.CompilerParams(
            dimension_semantics=("parallel",),
        ),
    )(m1q, aux, b2a, xq)

    return out.reshape(B, 1)


# speedup vs baseline: 17.8939x; 1.7981x over previous
import jax
import jax.numpy as jnp
from jax.experimental import pallas as pl
from jax.experimental.pallas import tpu as pltpu

_SD_PAD = 8


def _mlp_body(m1q_ref, b1p_ref, w2p_ref, b2_ref, x_ref, o_ref, *,
              hidden, n_dots, s_rows):
    chunks = s_rows // 4
    b2 = b2_ref[0, 0]
    b1p = b1p_ref[...]
    w2p = w2p_ref[...]
    for k in range(n_dots):
        xs = x_ref[pl.ds(k * s_rows, s_rows), :]
        h = jnp.dot(m1q_ref[...], xs, preferred_element_type=jnp.float32)
        g = jnp.maximum(h + b1p, 0.0) * w2p
        s = g.reshape(hidden, chunks, 128).sum(axis=0)
        o_ref[pl.ds(k * chunks, chunks), :] = s + b2


def kernel(x, params):
    B, sd = x.shape
    p_rows, hidden = params.shape
    assert p_rows == hidden + 1
    lanes = 128
    assert B % (lanes * 64) == 0

    w1 = params[:hidden, :sd]
    b1 = params[:hidden, sd]
    b2 = params[0, _SD_PAD]
    w2 = params[hidden, :hidden]

    nrows = (B // lanes) * sd
    xq = x.reshape(B // lanes, lanes, sd).transpose(0, 2, 1).reshape(nrows, lanes)

    s_rows = 256
    dot_chunks = s_rows // sd
    n_dots = 16
    while (B // lanes) % (n_dots * dot_chunks):
        n_dots //= 2
    step_rows = n_dots * s_rows
    grid = (nrows // step_rows,)

    eye = jnp.eye(dot_chunks, dtype=jnp.float32)
    m1q = (w1[:, None, None, :] * eye[None, :, :, None]).reshape(
        hidden * dot_chunks, dot_chunks * sd)
    rows = hidden * dot_chunks
    b1p = jnp.broadcast_to(
        b1[:, None, None], (hidden, dot_chunks, lanes)).reshape(rows, lanes)
    w2p = jnp.broadcast_to(
        w2[:, None, None], (hidden, dot_chunks, lanes)).reshape(rows, lanes)
    b2a = jnp.reshape(b2, (1, 1))

    out = pl.pallas_call(
        lambda m, bb, ww, b, xx, o: _mlp_body(
            m, bb, ww, b, xx, o, hidden=hidden, n_dots=n_dots, s_rows=s_rows),
        grid=grid,
        in_specs=[
            pl.BlockSpec((rows, s_rows), lambda i: (0, 0)),
            pl.BlockSpec((rows, lanes), lambda i: (0, 0)),
            pl.BlockSpec((rows, lanes), lambda i: (0, 0)),
            pl.BlockSpec(memory_space=pltpu.MemorySpace.SMEM),
            pl.BlockSpec((step_rows, lanes), lambda i: (i, 0)),
        ],
        out_specs=pl.BlockSpec((step_rows // sd, lanes), lambda i: (i, 0)),
        out_shape=jax.ShapeDtypeStruct((B // lanes, lanes), jnp.float32),
        compiler_params=pltpu.CompilerParams(
            dimension_semantics=("parallel",),
        ),
    )(m1q, b1p, w2p, b2a, xq)

    return out.reshape(B, 1)


# speedup vs baseline: 24.5305x; 1.3709x over previous
import jax
import jax.numpy as jnp
from jax.experimental import pallas as pl
from jax.experimental.pallas import tpu as pltpu

_SD_PAD = 8


def _mlp_body(m1p_ref, b1p_ref, w2p_ref, b2_ref, x_ref, o_ref, *,
              hidden, n_dots, s_rows):
    pairs = s_rows // 8
    b2 = b2_ref[0, 0]
    b1p = b1p_ref[...]
    w2p = w2p_ref[...]
    m1p = m1p_ref[...]
    left = (jax.lax.broadcasted_iota(jnp.int32, (s_rows, 128), 0) % 8) < 4
    zero = jnp.zeros((), jnp.float32)
    for k in range(n_dots):
        xs = x_ref[pl.ds(k * s_rows, s_rows), :]
        wide = jnp.concatenate(
            [jnp.where(left, xs, zero), jnp.where(left, zero, xs)], axis=1)
        h = jnp.dot(m1p, wide, preferred_element_type=jnp.float32)
        g = jnp.maximum(h + b1p, 0.0) * w2p
        s = g.reshape(hidden, pairs, 256).sum(axis=0)
        o_ref[pl.ds(k * pairs, pairs), :] = s + b2


def kernel(x, params):
    B, sd = x.shape
    p_rows, hidden = params.shape
    assert p_rows == hidden + 1
    lanes = 128
    assert B % (lanes * 16) == 0

    w1 = params[:hidden, :sd]
    b1 = params[:hidden, sd]
    b2 = params[0, _SD_PAD]
    w2 = params[hidden, :hidden]

    nrows = (B // lanes) * sd
    xq = x.reshape(B // lanes, lanes, sd).transpose(0, 2, 1).reshape(nrows, lanes)

    s_rows = 256
    pairs = s_rows // (2 * sd)
    n_dots = 16
    while (B // lanes) % (n_dots * 2 * pairs):
        n_dots //= 2
    step_rows = n_dots * s_rows
    grid = (nrows // step_rows,)

    rows = hidden * pairs
    eye = jnp.eye(pairs, dtype=jnp.float32)
    w1dup = jnp.concatenate([w1, w1], axis=1)
    m1p = (w1dup[:, None, None, :] * eye[None, :, :, None]).reshape(
        rows, pairs * 2 * sd)
    b1p = jnp.broadcast_to(
        b1[:, None, None], (hidden, pairs, 2 * lanes)).reshape(rows, 2 * lanes)
    w2p = jnp.broadcast_to(
        w2[:, None, None], (hidden, pairs, 2 * lanes)).reshape(rows, 2 * lanes)
    b2a = jnp.reshape(b2, (1, 1))

    out = pl.pallas_call(
        lambda m, bb, ww, b, xx, o: _mlp_body(
            m, bb, ww, b, xx, o, hidden=hidden, n_dots=n_dots, s_rows=s_rows),
        grid=grid,
        in_specs=[
            pl.BlockSpec((rows, pairs * 2 * sd), lambda i: (0, 0)),
            pl.BlockSpec((rows, 2 * lanes), lambda i: (0, 0)),
            pl.BlockSpec((rows, 2 * lanes), lambda i: (0, 0)),
            pl.BlockSpec(memory_space=pltpu.MemorySpace.SMEM),
            pl.BlockSpec((step_rows, lanes), lambda i: (i, 0)),
        ],
        out_specs=pl.BlockSpec((n_dots * pairs, 2 * lanes), lambda i: (i, 0)),
        out_shape=jax.ShapeDtypeStruct((B // (2 * lanes), 2 * lanes), jnp.float32),
        compiler_params=pltpu.CompilerParams(
            dimension_semantics=("parallel",),
        ),
    )(m1p, b1p, w2p, b2a, xq)

    return out.reshape(B, 1)


# speedup vs baseline: 24.7322x; 1.0082x over previous
import jax
import jax.numpy as jnp
from jax.experimental import pallas as pl
from jax.experimental.pallas import tpu as pltpu

_SD_PAD = 8


def _mlp_body(m1p_ref, b1p_ref, w2p_ref, b2_ref, x_ref, o_ref, *,
              hidden, n_dots, s_rows):
    pairs = s_rows // 8
    b2 = b2_ref[0, 0]
    b1p = b1p_ref[...]
    w2p = w2p_ref[...]
    m1p = m1p_ref[...]
    left = (jax.lax.broadcasted_iota(jnp.int32, (s_rows, 128), 0) % 8) < 4
    zero = jnp.zeros((), jnp.float32)
    for k in range(n_dots):
        xs = x_ref[pl.ds(k * s_rows, s_rows), :]
        wide = jnp.concatenate(
            [jnp.where(left, xs, zero), jnp.where(left, zero, xs)], axis=1)
        h = jnp.dot(m1p, wide, preferred_element_type=jnp.float32)
        g = jnp.maximum(h + b1p, 0.0) * w2p
        s = g.reshape(hidden, pairs, 256).sum(axis=0)
        o_ref[pl.ds(k * pairs, pairs), :] = s + b2


def kernel(x, params):
    B, sd = x.shape
    p_rows, hidden = params.shape
    assert p_rows == hidden + 1
    lanes = 128
    assert B % (lanes * 16) == 0

    w1 = params[:hidden, :sd]
    b1 = params[:hidden, sd]
    b2 = params[0, _SD_PAD]
    w2 = params[hidden, :hidden]

    nrows = (B // lanes) * sd
    xq = x.reshape(B // lanes, lanes, sd).transpose(0, 2, 1).reshape(nrows, lanes)

    s_rows = 256
    pairs = s_rows // (2 * sd)
    n_dots = 32
    while (B // lanes) % (n_dots * 2 * pairs):
        n_dots //= 2
    step_rows = n_dots * s_rows
    grid = (nrows // step_rows,)

    rows = hidden * pairs
    eye = jnp.eye(pairs, dtype=jnp.float32)
    w1dup = jnp.concatenate([w1, w1], axis=1)
    m1p = (w1dup[:, None, None, :] * eye[None, :, :, None]).reshape(
        rows, pairs * 2 * sd)
    b1p = jnp.broadcast_to(
        b1[:, None, None], (hidden, pairs, 2 * lanes)).reshape(rows, 2 * lanes)
    w2p = jnp.broadcast_to(
        w2[:, None, None], (hidden, pairs, 2 * lanes)).reshape(rows, 2 * lanes)
    b2a = jnp.reshape(b2, (1, 1))

    out = pl.pallas_call(
        lambda m, bb, ww, b, xx, o: _mlp_body(
            m, bb, ww, b, xx, o, hidden=hidden, n_dots=n_dots, s_rows=s_rows),
        grid=grid,
        in_specs=[
            pl.BlockSpec((rows, pairs * 2 * sd), lambda i: (0, 0)),
            pl.BlockSpec((rows, 2 * lanes), lambda i: (0, 0)),
            pl.BlockSpec((rows, 2 * lanes), lambda i: (0, 0)),
            pl.BlockSpec(memory_space=pltpu.MemorySpace.SMEM),
            pl.BlockSpec((step_rows, lanes), lambda i: (i, 0)),
        ],
        out_specs=pl.BlockSpec((n_dots * pairs, 2 * lanes), lambda i: (i, 0)),
        out_shape=jax.ShapeDtypeStruct((B // (2 * lanes), 2 * lanes), jnp.float32),
        compiler_params=pltpu.CompilerParams(
            dimension_semantics=("parallel",),
        ),
    )(m1p, b1p, w2p, b2a, xq)

    return out.reshape(B, 1)
